# baseline (speedup 1.0000x reference)
import jax
import jax.numpy as jnp
from jax import lax
from jax.experimental import pallas as pl
from jax.experimental.pallas import tpu as pltpu
from jax.experimental.pallas import tpu_sc as plsc

NU = 25000
NI = 25000
NN = NU + NI
D = 64
HD = 32
E = 800000
B = 16384

NS = 16
NPAD = 51200
RPT = NPAD // NS
CHUNK = 512
NCH = 98
ET = NCH * CHUNK
EPAD = NS * ET
PB = B // NS
PCH = 64


def _sc_body(t0, srcp, dstp, ew, uidx, iidx,
             usum, isum, t1, t2,
             acc, rows, srcgA, srcgB,
             dstA0, dstA1, dstA2, dstA3, dstB0, dstB1, dstB2, dstB3,
             wvA, wvB, ubuf, ibuf,
             gsem, ssem, lsem):
    c = lax.axis_index("c")
    s = lax.axis_index("s")
    zeros16 = jnp.zeros((16,), jnp.float32)
    i16 = lax.iota(jnp.int32, 16)
    zb = rows.at[pl.ds(0, 128)]
    srcgs = (srcgA, srcgB)
    wvs = (wvA, wvB)
    dstvs = ((dstA0, dstA1, dstA2, dstA3), (dstB0, dstB1, dstB2, dstB3))

    def zero_zb():
        def zb_body(i, carry):
            rows[i, pl.ds(0, 16)] = zeros16
            rows[i, pl.ds(16, 16)] = zeros16
            return carry

        lax.fori_loop(0, 128, zb_body, 0)

    zero_zb()
    zdescs = [
        pltpu.async_copy(
            zb, acc.at[pl.ds(pl.multiple_of(s * RPT + r * 128, 128), 128)],
            gsem)
        for r in range(RPT // 128)
    ]
    for dsc in zdescs:
        dsc.wait()
    plsc.subcore_barrier()

    def layer(tbl_in):
        def fire_loads(ch, p):
            ebase = pl.multiple_of(s * ET + ch * CHUNK, CHUNK)
            sbase = pl.multiple_of(c * EPAD + s * ET + ch * CHUNK, CHUNK)
            pltpu.async_copy(srcp.at[pl.ds(sbase, CHUNK)], srcgs[p], lsem)
            pltpu.async_copy(ew.at[pl.ds(ebase, CHUNK)], wvs[p], lsem)
            for j in range(4):
                pltpu.async_copy(
                    dstp.at[pl.ds(pl.multiple_of(ebase + j * 128, 128), 128)],
                    dstvs[p][j], lsem)

        def drain_loads(ch, p):
            ebase = pl.multiple_of(s * ET + ch * CHUNK, CHUNK)
            sbase = pl.multiple_of(c * EPAD + s * ET + ch * CHUNK, CHUNK)
            pltpu.make_async_copy(
                srcp.at[pl.ds(sbase, CHUNK)], srcgs[p], lsem).wait()
            pltpu.make_async_copy(
                ew.at[pl.ds(ebase, CHUNK)], wvs[p], lsem).wait()
            for j in range(4):
                pltpu.make_async_copy(
                    dstp.at[pl.ds(pl.multiple_of(ebase + j * 128, 128), 128)],
                    dstvs[p][j], lsem).wait()

        def chunk_step(ch, p):
            drain_loads(ch, p)
            descs = [
                pltpu.async_copy(
                    tbl_in.at[srcgs[p].at[pl.ds(j * 128, 128)]],
                    rows.at[pl.ds(j * 128, 128)], gsem)
                for j in range(4)
            ]
            nxt = jnp.minimum(ch + 1, NCH - 1)
            fire_loads(nxt, 1 - p)
            for j in range(4):
                descs[j].wait()

                def mul_body(g, cr):
                    base = pl.multiple_of(j * 128 + g * 16, 16)
                    rid = base + i16
                    wvv = wvs[p][pl.ds(base, 16)]
                    for d in range(HD):
                        fd = jnp.full((16,), d, jnp.int32)
                        v = plsc.load_gather(rows, [rid, fd])
                        plsc.store_scatter(rows, [rid, fd], v * wvv)
                    return cr

                lax.fori_loop(0, 8, mul_body, 0)
                pltpu.sync_copy(rows.at[pl.ds(j * 128, 128)],
                                acc.at[dstvs[p][j]], add=True)

        fire_loads(0, 0)

        def pair_step(i, carry):
            chunk_step(2 * i, 0)
            chunk_step(2 * i + 1, 1)
            return carry

        lax.fori_loop(0, NCH // 2, pair_step, 0)
        drain_loads(NCH - 1, 0)
        plsc.subcore_barrier()

    def writeback(dst_tbl, rezero):
        base = s * RPT
        coff = c * NPAD
        wdescs = [
            pltpu.async_copy(
                acc.at[pl.ds(pl.multiple_of(base + r * 128, 128), 128)],
                dst_tbl.at[pl.ds(pl.multiple_of(coff + base + r * 128, 128),
                                 128)],
                ssem)
            for r in range(RPT // 128)
        ]
        for dsc in wdescs:
            dsc.wait()
        if rezero:
            zero_zb()
            zdescs = [
                pltpu.async_copy(
                    zb,
                    acc.at[pl.ds(pl.multiple_of(base + r * 128, 128), 128)],
                    gsem)
                for r in range(RPT // 128)
            ]
            for dsc in zdescs:
                dsc.wait()
        plsc.subcore_barrier()

    layer(t0)
    writeback(t1, rezero=True)
    layer(t1)
    writeback(t2, rezero=False)

    def pair_body(q, cr):
        qs = pl.multiple_of(s * PB + q * PCH, PCH)
        gb = pl.multiple_of(c * B + s * PB + q * PCH, PCH)
        l1 = pltpu.async_copy(uidx.at[pl.ds(gb, PCH)], ubuf, gsem)
        l2 = pltpu.async_copy(iidx.at[pl.ds(gb, PCH)], ibuf, gsem)
        l1.wait()
        l2.wait()

        descs = [
            pltpu.async_copy(t0.at[ubuf], rows.at[pl.ds(0, PCH)], gsem),
            pltpu.async_copy(t1.at[ubuf], rows.at[pl.ds(64, PCH)], gsem),
            pltpu.async_copy(t2.at[ubuf], rows.at[pl.ds(128, PCH)], gsem),
            pltpu.async_copy(t0.at[ibuf], rows.at[pl.ds(192, PCH)], gsem),
            pltpu.async_copy(t1.at[ibuf], rows.at[pl.ds(256, PCH)], gsem),
            pltpu.async_copy(t2.at[ibuf], rows.at[pl.ds(320, PCH)], gsem),
        ]
        for dsc in descs:
            dsc.wait()

        def sum_body(g, cr2):
            gb = g * 16
            for j in range(16):
                p = gb + j
                for h in (0, 16):
                    sl = pl.ds(h, 16)
                    rows[p, sl] = (rows[p, sl] + rows[64 + p, sl]
                                   + rows[128 + p, sl])
                    rows[192 + p, sl] = (rows[192 + p, sl]
                                         + rows[256 + p, sl]
                                         + rows[320 + p, sl])
            return cr2

        lax.fori_loop(0, PCH // 16, sum_body, 0)
        pltpu.sync_copy(rows.at[pl.ds(0, PCH)], usum.at[c, pl.ds(qs, PCH)])
        pltpu.sync_copy(rows.at[pl.ds(192, PCH)], isum.at[c, pl.ds(qs, PCH)])
        return cr

    lax.fori_loop(0, PB // PCH, pair_body, 0)


_sc_call = pl.kernel(
    _sc_body,
    out_type=[
        jax.ShapeDtypeStruct((2, B, HD), jnp.float32),
        jax.ShapeDtypeStruct((2, B, HD), jnp.float32),
        jax.ShapeDtypeStruct((2 * NPAD, HD), jnp.float32),
        jax.ShapeDtypeStruct((2 * NPAD, HD), jnp.float32),
    ],
    mesh=plsc.VectorSubcoreMesh(core_axis_name="c", subcore_axis_name="s"),
    compiler_params=pltpu.CompilerParams(use_tc_tiling_on_sc=False, needs_layout_passes=False),
    scratch_types=[
        pltpu.VMEM_SHARED((NPAD, HD), jnp.float32),
        pltpu.VMEM((CHUNK, HD), jnp.float32),
        pltpu.VMEM((CHUNK,), jnp.int32),
        pltpu.VMEM((CHUNK,), jnp.int32),
        pltpu.VMEM((128,), jnp.int32),
        pltpu.VMEM((128,), jnp.int32),
        pltpu.VMEM((128,), jnp.int32),
        pltpu.VMEM((128,), jnp.int32),
        pltpu.VMEM((128,), jnp.int32),
        pltpu.VMEM((128,), jnp.int32),
        pltpu.VMEM((128,), jnp.int32),
        pltpu.VMEM((128,), jnp.int32),
        pltpu.VMEM((CHUNK,), jnp.float32),
        pltpu.VMEM((CHUNK,), jnp.float32),
        pltpu.VMEM((PCH,), jnp.int32),
        pltpu.VMEM((PCH,), jnp.int32),
        pltpu.SemaphoreType.DMA,
        pltpu.SemaphoreType.DMA,
        pltpu.SemaphoreType.DMA,
    ],
)


def _loss_body(u_ref, i_ref, y_ref, out_ref):
    m = u_ref[0] * i_ref[0] + u_ref[1] * i_ref[1]
    l = jnp.sum(m, axis=-1) * (1.0 / 9.0)
    y = y_ref[...]
    sp = jnp.maximum(l, 0.0) + jnp.log1p(jnp.exp(-jnp.abs(l)))
    out_ref[0, 0] = jnp.sum(sp - l * y) * (1.0 / B)


_loss_call = pl.pallas_call(
    _loss_body,
    out_shape=jax.ShapeDtypeStruct((1, 1), jnp.float32),
    in_specs=[
        pl.BlockSpec(memory_space=pltpu.VMEM),
        pl.BlockSpec(memory_space=pltpu.VMEM),
        pl.BlockSpec(memory_space=pltpu.VMEM),
    ],
    out_specs=pl.BlockSpec(memory_space=pltpu.SMEM),
)


def kernel(user_emb, item_emb, edge_weight, users, items, labels, edge_index):
    all_emb = jnp.concatenate([user_emb, item_emb], axis=0)
    allp = jnp.concatenate(
        [all_emb, jnp.zeros((NPAD - NN, D), jnp.float32)], axis=0)
    t0 = jnp.concatenate([allp[:, :HD], allp[:, HD:]], axis=0)

    src_p0 = jnp.concatenate(
        [edge_index[0], jnp.zeros((EPAD - E,), jnp.int32)])
    src_p = jnp.concatenate([src_p0, src_p0 + NPAD])
    dst_p = jnp.concatenate(
        [edge_index[1], jnp.full((EPAD - E,), NN, jnp.int32)])
    ew_p = jnp.concatenate(
        [edge_weight, jnp.zeros((EPAD - E,), jnp.float32)])

    users_b = jnp.concatenate([users, users + NPAD])
    items_b = jnp.concatenate([items + NU, items + (NU + NPAD)])

    usum, isum, _, _ = _sc_call(t0, src_p, dst_p, ew_p, users_b, items_b)

    u4 = usum.reshape(2, 128, 128, HD)
    i4 = isum.reshape(2, 128, 128, HD)
    y = labels.astype(jnp.float32).reshape(128, 128)
    loss = _loss_call(u4, i4, y)
    return loss[0, 0]


# speedup vs baseline: 6.1537x; 6.1537x over previous
import jax
import jax.numpy as jnp
from jax import lax
from jax.experimental import pallas as pl
from jax.experimental.pallas import tpu as pltpu
from jax.experimental.pallas import tpu_sc as plsc

NU = 25000
NI = 25000
NN = NU + NI
D = 64
HD = 32
E = 800000
B = 16384

NS = 16
NPAD = 51200
RPT = NPAD // NS
CHUNK = 512
NCH = 98
ET = NCH * CHUNK
EPAD = NS * ET
PB = B // NS
PCH = 64


def _sc_body(t0, srcp, dstp, ew, uidx, iidx,
             usum, isum, t1, t2,
             acc, rows, srcgA, srcgB,
             dstA0, dstA1, dstA2, dstA3, dstB0, dstB1, dstB2, dstB3,
             wvA, wvB, ubuf, ibuf,
             gsem, ssem, lsem):
    c = lax.axis_index("c")
    s = lax.axis_index("s")
    zeros16 = jnp.zeros((16,), jnp.float32)
    zb = rows.at[pl.ds(0, 128)]
    srcgs = (srcgA, srcgB)
    wvs = (wvA, wvB)
    dstvs = ((dstA0, dstA1, dstA2, dstA3), (dstB0, dstB1, dstB2, dstB3))

    def zero_zb():
        def zb_body(i, carry):
            rows[i, pl.ds(0, 16)] = zeros16
            rows[i, pl.ds(16, 16)] = zeros16
            return carry

        lax.fori_loop(0, 128, zb_body, 0)

    zero_zb()
    zdescs = [
        pltpu.async_copy(
            zb, acc.at[pl.ds(pl.multiple_of(s * RPT + r * 128, 128), 128)],
            gsem)
        for r in range(RPT // 128)
    ]
    for dsc in zdescs:
        dsc.wait()
    plsc.subcore_barrier()

    def layer(tbl_in):
        def fire_loads(ch, p):
            ebase = pl.multiple_of(s * ET + ch * CHUNK, CHUNK)
            sbase = pl.multiple_of(c * EPAD + s * ET + ch * CHUNK, CHUNK)
            pltpu.async_copy(srcp.at[pl.ds(sbase, CHUNK)], srcgs[p], lsem)
            pltpu.async_copy(ew.at[pl.ds(ebase, CHUNK)], wvs[p], lsem)
            for j in range(4):
                pltpu.async_copy(
                    dstp.at[pl.ds(pl.multiple_of(ebase + j * 128, 128), 128)],
                    dstvs[p][j], lsem)

        def drain_loads(ch, p):
            ebase = pl.multiple_of(s * ET + ch * CHUNK, CHUNK)
            sbase = pl.multiple_of(c * EPAD + s * ET + ch * CHUNK, CHUNK)
            pltpu.make_async_copy(
                srcp.at[pl.ds(sbase, CHUNK)], srcgs[p], lsem).wait()
            pltpu.make_async_copy(
                ew.at[pl.ds(ebase, CHUNK)], wvs[p], lsem).wait()
            for j in range(4):
                pltpu.make_async_copy(
                    dstp.at[pl.ds(pl.multiple_of(ebase + j * 128, 128), 128)],
                    dstvs[p][j], lsem).wait()

        def drain_scatters(p):
            for j in range(4):
                pltpu.make_async_copy(rows.at[pl.ds(j * 128, 128)],
                                      acc.at[dstvs[p][j]], ssem).wait()

        def chunk_step(ch, p, first=False):
            drain_loads(ch, p)
            if not first:
                drain_scatters(1 - p)
            descs = [
                pltpu.async_copy(
                    tbl_in.at[srcgs[p].at[pl.ds(j * 128, 128)]],
                    rows.at[pl.ds(j * 128, 128)], gsem)
                for j in range(4)
            ]
            nxt = jnp.minimum(ch + 1, NCH - 1)
            fire_loads(nxt, 1 - p)
            for j in range(4):
                descs[j].wait()

                def mul_body(g, cr):
                    base = pl.multiple_of(j * 128 + g * 16, 16)
                    wvv = wvs[p][pl.ds(base, 16)]
                    for jj in range(16):
                        e = base + jj
                        wj = jnp.full((16,), wvv[jj], jnp.float32)
                        a = rows[e, pl.ds(0, 16)]
                        rows[e, pl.ds(0, 16)] = a * wj
                        b = rows[e, pl.ds(16, 16)]
                        rows[e, pl.ds(16, 16)] = b * wj
                    return cr

                lax.fori_loop(0, 8, mul_body, 0)
                pltpu.async_copy(rows.at[pl.ds(j * 128, 128)],
                                 acc.at[dstvs[p][j]], ssem, add=True)

        fire_loads(0, 0)
        chunk_step(0, 0, first=True)

        def pair_step(i, carry):
            chunk_step(2 * i + 1, 1)
            chunk_step(2 * i + 2, 0)
            return carry

        lax.fori_loop(0, (NCH - 2) // 2, pair_step, 0)
        chunk_step(NCH - 1, 1)
        drain_scatters(1)
        drain_loads(NCH - 1, 0)
        plsc.subcore_barrier()

    def writeback(dst_tbl, rezero):
        base = s * RPT
        coff = c * NPAD
        wdescs = [
            pltpu.async_copy(
                acc.at[pl.ds(pl.multiple_of(base + r * 128, 128), 128)],
                dst_tbl.at[pl.ds(pl.multiple_of(coff + base + r * 128, 128),
                                 128)],
                ssem)
            for r in range(RPT // 128)
        ]
        for dsc in wdescs:
            dsc.wait()
        if rezero:
            zero_zb()
            zdescs = [
                pltpu.async_copy(
                    zb,
                    acc.at[pl.ds(pl.multiple_of(base + r * 128, 128), 128)],
                    gsem)
                for r in range(RPT // 128)
            ]
            for dsc in zdescs:
                dsc.wait()
        plsc.subcore_barrier()

    layer(t0)
    writeback(t1, rezero=True)
    layer(t1)
    writeback(t2, rezero=False)

    def pair_body(q, cr):
        qs = pl.multiple_of(s * PB + q * PCH, PCH)
        gb = pl.multiple_of(c * B + s * PB + q * PCH, PCH)
        l1 = pltpu.async_copy(uidx.at[pl.ds(gb, PCH)], ubuf, gsem)
        l2 = pltpu.async_copy(iidx.at[pl.ds(gb, PCH)], ibuf, gsem)
        l1.wait()
        l2.wait()

        descs = [
            pltpu.async_copy(t0.at[ubuf], rows.at[pl.ds(0, PCH)], gsem),
            pltpu.async_copy(t1.at[ubuf], rows.at[pl.ds(64, PCH)], gsem),
            pltpu.async_copy(t2.at[ubuf], rows.at[pl.ds(128, PCH)], gsem),
            pltpu.async_copy(t0.at[ibuf], rows.at[pl.ds(192, PCH)], gsem),
            pltpu.async_copy(t1.at[ibuf], rows.at[pl.ds(256, PCH)], gsem),
            pltpu.async_copy(t2.at[ibuf], rows.at[pl.ds(320, PCH)], gsem),
        ]
        for dsc in descs:
            dsc.wait()

        def sum_body(g, cr2):
            gb = g * 16
            for j in range(16):
                p = gb + j
                for h in (0, 16):
                    sl = pl.ds(h, 16)
                    rows[p, sl] = (rows[p, sl] + rows[64 + p, sl]
                                   + rows[128 + p, sl])
                    rows[192 + p, sl] = (rows[192 + p, sl]
                                         + rows[256 + p, sl]
                                         + rows[320 + p, sl])
            return cr2

        lax.fori_loop(0, PCH // 16, sum_body, 0)
        pltpu.sync_copy(rows.at[pl.ds(0, PCH)], usum.at[c, pl.ds(qs, PCH)])
        pltpu.sync_copy(rows.at[pl.ds(192, PCH)], isum.at[c, pl.ds(qs, PCH)])
        return cr

    lax.fori_loop(0, PB // PCH, pair_body, 0)


_sc_call = pl.kernel(
    _sc_body,
    out_type=[
        jax.ShapeDtypeStruct((2, B, HD), jnp.float32),
        jax.ShapeDtypeStruct((2, B, HD), jnp.float32),
        jax.ShapeDtypeStruct((2 * NPAD, HD), jnp.float32),
        jax.ShapeDtypeStruct((2 * NPAD, HD), jnp.float32),
    ],
    mesh=plsc.VectorSubcoreMesh(core_axis_name="c", subcore_axis_name="s"),
    compiler_params=pltpu.CompilerParams(use_tc_tiling_on_sc=False),
    scratch_types=[
        pltpu.VMEM_SHARED((NPAD, HD), jnp.float32),
        pltpu.VMEM((CHUNK, HD), jnp.float32),
        pltpu.VMEM((CHUNK,), jnp.int32),
        pltpu.VMEM((CHUNK,), jnp.int32),
        pltpu.VMEM((128,), jnp.int32),
        pltpu.VMEM((128,), jnp.int32),
        pltpu.VMEM((128,), jnp.int32),
        pltpu.VMEM((128,), jnp.int32),
        pltpu.VMEM((128,), jnp.int32),
        pltpu.VMEM((128,), jnp.int32),
        pltpu.VMEM((128,), jnp.int32),
        pltpu.VMEM((128,), jnp.int32),
        pltpu.VMEM((CHUNK,), jnp.float32),
        pltpu.VMEM((CHUNK,), jnp.float32),
        pltpu.VMEM((PCH,), jnp.int32),
        pltpu.VMEM((PCH,), jnp.int32),
        pltpu.SemaphoreType.DMA,
        pltpu.SemaphoreType.DMA,
        pltpu.SemaphoreType.DMA,
    ],
)


def _loss_body(u_ref, i_ref, y_ref, out_ref):
    m = u_ref[0] * i_ref[0] + u_ref[1] * i_ref[1]
    l = jnp.sum(m, axis=-1) * (1.0 / 9.0)
    y = y_ref[...]
    sp = jnp.maximum(l, 0.0) + jnp.log1p(jnp.exp(-jnp.abs(l)))
    out_ref[0, 0] = jnp.sum(sp - l * y) * (1.0 / B)


_loss_call = pl.pallas_call(
    _loss_body,
    out_shape=jax.ShapeDtypeStruct((1, 1), jnp.float32),
    in_specs=[
        pl.BlockSpec(memory_space=pltpu.VMEM),
        pl.BlockSpec(memory_space=pltpu.VMEM),
        pl.BlockSpec(memory_space=pltpu.VMEM),
    ],
    out_specs=pl.BlockSpec(memory_space=pltpu.SMEM),
)


def kernel(user_emb, item_emb, edge_weight, users, items, labels, edge_index):
    all_emb = jnp.concatenate([user_emb, item_emb], axis=0)
    allp = jnp.concatenate(
        [all_emb, jnp.zeros((NPAD - NN, D), jnp.float32)], axis=0)
    t0 = jnp.concatenate([allp[:, :HD], allp[:, HD:]], axis=0)

    src_p0 = jnp.concatenate(
        [edge_index[0], jnp.zeros((EPAD - E,), jnp.int32)])
    src_p = jnp.concatenate([src_p0, src_p0 + NPAD])
    dst_p = jnp.concatenate(
        [edge_index[1], jnp.full((EPAD - E,), NN, jnp.int32)])
    ew_p = jnp.concatenate(
        [edge_weight, jnp.zeros((EPAD - E,), jnp.float32)])

    users_b = jnp.concatenate([users, users + NPAD])
    items_b = jnp.concatenate([items + NU, items + (NU + NPAD)])

    usum, isum, _, _ = _sc_call(t0, src_p, dst_p, ew_p, users_b, items_b)

    u4 = usum.reshape(2, 128, 128, HD)
    i4 = isum.reshape(2, 128, 128, HD)
    y = labels.astype(jnp.float32).reshape(128, 128)
    loss = _loss_call(u4, i4, y)
    return loss[0, 0]


# speedup vs baseline: 6.4972x; 1.0558x over previous
import jax
import jax.numpy as jnp
from jax import lax
from jax.experimental import pallas as pl
from jax.experimental.pallas import tpu as pltpu
from jax.experimental.pallas import tpu_sc as plsc

NU = 25000
NI = 25000
NN = NU + NI
D = 64
HD = 32
E = 800000
B = 16384

NS = 16
NPAD = 51200
RPT = NPAD // NS
CHUNK = 512
NCH = 98
ET = NCH * CHUNK
EPAD = NS * ET
PB = B // NS
PCH = 64


def _sc_body(t0, srcp, dstp, ew, uidx, iidx,
             usum, isum, t1, t2,
             acc, rows, srcgA, srcgB,
             dstA0, dstA1, dstA2, dstA3, dstB0, dstB1, dstB2, dstB3,
             wvA, wvB, ubuf, ibuf,
             gsem, ssem, lsem):
    c = lax.axis_index("c")
    s = lax.axis_index("s")
    zeros16 = jnp.zeros((16,), jnp.float32)
    zb = rows.at[pl.ds(0, 128)]
    srcgs = (srcgA, srcgB)
    wvs = (wvA, wvB)
    dstvs = ((dstA0, dstA1, dstA2, dstA3), (dstB0, dstB1, dstB2, dstB3))

    def zero_zb():
        def zb_body(i, carry):
            rows[i, pl.ds(0, 16)] = zeros16
            rows[i, pl.ds(16, 16)] = zeros16
            return carry

        lax.fori_loop(0, 128, zb_body, 0)

    zero_zb()
    zdescs = [
        pltpu.async_copy(
            zb, acc.at[pl.ds(pl.multiple_of(s * RPT + r * 128, 128), 128)],
            gsem)
        for r in range(RPT // 128)
    ]
    for dsc in zdescs:
        dsc.wait()
    plsc.subcore_barrier()

    def layer(tbl_in):
        def fire_loads(ch, p):
            ebase = pl.multiple_of(s * ET + ch * CHUNK, CHUNK)
            sbase = pl.multiple_of(c * EPAD + s * ET + ch * CHUNK, CHUNK)
            pltpu.async_copy(srcp.at[pl.ds(sbase, CHUNK)], srcgs[p], lsem)
            pltpu.async_copy(ew.at[pl.ds(ebase, CHUNK)], wvs[p], lsem)
            for j in range(4):
                pltpu.async_copy(
                    dstp.at[pl.ds(pl.multiple_of(ebase + j * 128, 128), 128)],
                    dstvs[p][j], lsem)

        def drain_loads(ch, p):
            ebase = pl.multiple_of(s * ET + ch * CHUNK, CHUNK)
            sbase = pl.multiple_of(c * EPAD + s * ET + ch * CHUNK, CHUNK)
            pltpu.make_async_copy(
                srcp.at[pl.ds(sbase, CHUNK)], srcgs[p], lsem).wait()
            pltpu.make_async_copy(
                ew.at[pl.ds(ebase, CHUNK)], wvs[p], lsem).wait()
            for j in range(4):
                pltpu.make_async_copy(
                    dstp.at[pl.ds(pl.multiple_of(ebase + j * 128, 128), 128)],
                    dstvs[p][j], lsem).wait()

        def fire_gather(p, j):
            pltpu.async_copy(tbl_in.at[srcgs[p].at[pl.ds(j * 128, 128)]],
                             rows.at[pl.ds(j * 128, 128)], gsem)

        def drain_gather(p, j):
            pltpu.make_async_copy(
                tbl_in.at[srcgs[p].at[pl.ds(j * 128, 128)]],
                rows.at[pl.ds(j * 128, 128)], gsem).wait()

        def fire_scatter(p, j):
            pltpu.async_copy(rows.at[pl.ds(j * 128, 128)],
                             acc.at[dstvs[p][j]], ssem, add=True)

        def drain_scatter(p, j):
            pltpu.make_async_copy(rows.at[pl.ds(j * 128, 128)],
                                  acc.at[dstvs[p][j]], ssem).wait()

        def chunk_step(ch, p):
            nxt1 = jnp.minimum(ch + 1, NCH - 1)
            drain_loads(nxt1, 1 - p)
            for j in range(4):
                drain_gather(p, j)

                @plsc.parallel_loop(0, 8, unroll=4)
                def mul_body(g):
                    base = pl.multiple_of(j * 128 + g * 16, 16)
                    wvv = wvs[p][pl.ds(base, 16)]
                    for jj in range(16):
                        e = base + jj
                        wj = jnp.full((16,), wvv[jj], jnp.float32)
                        a = rows[e, pl.ds(0, 16)]
                        rows[e, pl.ds(0, 16)] = a * wj
                        b = rows[e, pl.ds(16, 16)]
                        rows[e, pl.ds(16, 16)] = b * wj

                fire_scatter(p, j)
                if j >= 1:
                    drain_scatter(p, j - 1)
                    fire_gather(1 - p, j - 1)
            drain_scatter(p, 3)
            fire_gather(1 - p, 3)
            fire_loads(jnp.minimum(ch + 2, NCH - 1), p)

        fire_loads(0, 0)
        drain_loads(0, 0)
        for j in range(4):
            fire_gather(0, j)
        fire_loads(1, 1)

        def pair_step(i, carry):
            chunk_step(2 * i, 0)
            chunk_step(2 * i + 1, 1)
            return carry

        lax.fori_loop(0, NCH // 2, pair_step, 0)
        for j in range(4):
            drain_gather(0, j)
        drain_loads(NCH - 1, 1)
        plsc.subcore_barrier()

    def writeback(dst_tbl, rezero):
        base = s * RPT
        coff = c * NPAD
        wdescs = [
            pltpu.async_copy(
                acc.at[pl.ds(pl.multiple_of(base + r * 128, 128), 128)],
                dst_tbl.at[pl.ds(pl.multiple_of(coff + base + r * 128, 128),
                                 128)],
                ssem)
            for r in range(RPT // 128)
        ]
        for dsc in wdescs:
            dsc.wait()
        if rezero:
            zero_zb()
            zdescs = [
                pltpu.async_copy(
                    zb,
                    acc.at[pl.ds(pl.multiple_of(base + r * 128, 128), 128)],
                    gsem)
                for r in range(RPT // 128)
            ]
            for dsc in zdescs:
                dsc.wait()
        plsc.subcore_barrier()

    layer(t0)
    writeback(t1, rezero=True)
    layer(t1)
    writeback(t2, rezero=False)

    def pair_body(q, cr):
        qs = pl.multiple_of(s * PB + q * PCH, PCH)
        gb = pl.multiple_of(c * B + s * PB + q * PCH, PCH)
        l1 = pltpu.async_copy(uidx.at[pl.ds(gb, PCH)], ubuf, gsem)
        l2 = pltpu.async_copy(iidx.at[pl.ds(gb, PCH)], ibuf, gsem)
        l1.wait()
        l2.wait()

        descs = [
            pltpu.async_copy(t0.at[ubuf], rows.at[pl.ds(0, PCH)], gsem),
            pltpu.async_copy(t1.at[ubuf], rows.at[pl.ds(64, PCH)], gsem),
            pltpu.async_copy(t2.at[ubuf], rows.at[pl.ds(128, PCH)], gsem),
            pltpu.async_copy(t0.at[ibuf], rows.at[pl.ds(192, PCH)], gsem),
            pltpu.async_copy(t1.at[ibuf], rows.at[pl.ds(256, PCH)], gsem),
            pltpu.async_copy(t2.at[ibuf], rows.at[pl.ds(320, PCH)], gsem),
        ]
        for dsc in descs:
            dsc.wait()

        def sum_body(g, cr2):
            gb = g * 16
            for j in range(16):
                p = gb + j
                for h in (0, 16):
                    sl = pl.ds(h, 16)
                    rows[p, sl] = (rows[p, sl] + rows[64 + p, sl]
                                   + rows[128 + p, sl])
                    rows[192 + p, sl] = (rows[192 + p, sl]
                                         + rows[256 + p, sl]
                                         + rows[320 + p, sl])
            return cr2

        lax.fori_loop(0, PCH // 16, sum_body, 0)
        pltpu.sync_copy(rows.at[pl.ds(0, PCH)], usum.at[c, pl.ds(qs, PCH)])
        pltpu.sync_copy(rows.at[pl.ds(192, PCH)], isum.at[c, pl.ds(qs, PCH)])
        return cr

    lax.fori_loop(0, PB // PCH, pair_body, 0)


_sc_call = pl.kernel(
    _sc_body,
    out_type=[
        jax.ShapeDtypeStruct((2, B, HD), jnp.float32),
        jax.ShapeDtypeStruct((2, B, HD), jnp.float32),
        jax.ShapeDtypeStruct((2 * NPAD, HD), jnp.float32),
        jax.ShapeDtypeStruct((2 * NPAD, HD), jnp.float32),
    ],
    mesh=plsc.VectorSubcoreMesh(core_axis_name="c", subcore_axis_name="s"),
    compiler_params=pltpu.CompilerParams(use_tc_tiling_on_sc=False),
    scratch_types=[
        pltpu.VMEM_SHARED((NPAD, HD), jnp.float32),
        pltpu.VMEM((CHUNK, HD), jnp.float32),
        pltpu.VMEM((CHUNK,), jnp.int32),
        pltpu.VMEM((CHUNK,), jnp.int32),
        pltpu.VMEM((128,), jnp.int32),
        pltpu.VMEM((128,), jnp.int32),
        pltpu.VMEM((128,), jnp.int32),
        pltpu.VMEM((128,), jnp.int32),
        pltpu.VMEM((128,), jnp.int32),
        pltpu.VMEM((128,), jnp.int32),
        pltpu.VMEM((128,), jnp.int32),
        pltpu.VMEM((128,), jnp.int32),
        pltpu.VMEM((CHUNK,), jnp.float32),
        pltpu.VMEM((CHUNK,), jnp.float32),
        pltpu.VMEM((PCH,), jnp.int32),
        pltpu.VMEM((PCH,), jnp.int32),
        pltpu.SemaphoreType.DMA,
        pltpu.SemaphoreType.DMA,
        pltpu.SemaphoreType.DMA,
    ],
)


def _loss_body(u_ref, i_ref, y_ref, out_ref):
    m = u_ref[0] * i_ref[0] + u_ref[1] * i_ref[1]
    l = jnp.sum(m, axis=-1) * (1.0 / 9.0)
    y = y_ref[...]
    sp = jnp.maximum(l, 0.0) + jnp.log1p(jnp.exp(-jnp.abs(l)))
    out_ref[0, 0] = jnp.sum(sp - l * y) * (1.0 / B)


_loss_call = pl.pallas_call(
    _loss_body,
    out_shape=jax.ShapeDtypeStruct((1, 1), jnp.float32),
    in_specs=[
        pl.BlockSpec(memory_space=pltpu.VMEM),
        pl.BlockSpec(memory_space=pltpu.VMEM),
        pl.BlockSpec(memory_space=pltpu.VMEM),
    ],
    out_specs=pl.BlockSpec(memory_space=pltpu.SMEM),
)


def kernel(user_emb, item_emb, edge_weight, users, items, labels, edge_index):
    all_emb = jnp.concatenate([user_emb, item_emb], axis=0)
    allp = jnp.concatenate(
        [all_emb, jnp.zeros((NPAD - NN, D), jnp.float32)], axis=0)
    t0 = jnp.concatenate([allp[:, :HD], allp[:, HD:]], axis=0)

    src_p0 = jnp.concatenate(
        [edge_index[0], jnp.zeros((EPAD - E,), jnp.int32)])
    src_p = jnp.concatenate([src_p0, src_p0 + NPAD])
    dst_p = jnp.concatenate(
        [edge_index[1], jnp.full((EPAD - E,), NN, jnp.int32)])
    ew_p = jnp.concatenate(
        [edge_weight, jnp.zeros((EPAD - E,), jnp.float32)])

    users_b = jnp.concatenate([users, users + NPAD])
    items_b = jnp.concatenate([items + NU, items + (NU + NPAD)])

    usum, isum, _, _ = _sc_call(t0, src_p, dst_p, ew_p, users_b, items_b)

    u4 = usum.reshape(2, 128, 128, HD)
    i4 = isum.reshape(2, 128, 128, HD)
    y = labels.astype(jnp.float32).reshape(128, 128)
    loss = _loss_call(u4, i4, y)
    return loss[0, 0]


# speedup vs baseline: 6.5598x; 1.0096x over previous
import jax
import jax.numpy as jnp
from jax import lax
from jax.experimental import pallas as pl
from jax.experimental.pallas import tpu as pltpu
from jax.experimental.pallas import tpu_sc as plsc

NU = 25000
NI = 25000
NN = NU + NI
D = 64
HD = 32
E = 800000
B = 16384

NS = 16
NPAD = 51200
RPT = NPAD // NS
CHUNK = 512
NCH = 98
ET = NCH * CHUNK
EPAD = NS * ET
PB = B // NS
PCH = 64


def _sc_body(t0, srcp, dstp, ew, uidx, iidx,
             usum, isum, t1, t2,
             acc, rows, srcgA, srcgB,
             dstA0, dstA1, dstA2, dstA3, dstB0, dstB1, dstB2, dstB3,
             wvA, wvB, ubuf, ibuf,
             gsem, ssem, lsem, psem):
    c = lax.axis_index("c")
    s = lax.axis_index("s")
    zeros16 = jnp.zeros((16,), jnp.float32)
    zb = rows.at[pl.ds(0, 128)]
    srcgs = (srcgA, srcgB)
    wvs = (wvA, wvB)
    dstvs = ((dstA0, dstA1, dstA2, dstA3), (dstB0, dstB1, dstB2, dstB3))

    def zero_zb():
        def zb_body(i, carry):
            rows[i, pl.ds(0, 16)] = zeros16
            rows[i, pl.ds(16, 16)] = zeros16
            return carry

        lax.fori_loop(0, 128, zb_body, 0)

    gb0 = pl.multiple_of(c * B + s * PB, PB)
    pltpu.async_copy(uidx.at[pl.ds(gb0, PB)], ubuf, psem)
    pltpu.async_copy(iidx.at[pl.ds(gb0, PB)], ibuf, psem)

    zero_zb()
    zdescs = [
        pltpu.async_copy(
            zb, acc.at[pl.ds(pl.multiple_of(s * RPT + r * 128, 128), 128)],
            gsem)
        for r in range(RPT // 128)
    ]
    for dsc in zdescs:
        dsc.wait()
    plsc.subcore_barrier()

    def layer(tbl_in):
        def fire_loads(ch, p):
            ebase = pl.multiple_of(s * ET + ch * CHUNK, CHUNK)
            sbase = pl.multiple_of(c * EPAD + s * ET + ch * CHUNK, CHUNK)
            pltpu.async_copy(srcp.at[pl.ds(sbase, CHUNK)], srcgs[p], lsem)
            pltpu.async_copy(ew.at[pl.ds(ebase, CHUNK)], wvs[p], lsem)
            for j in range(4):
                pltpu.async_copy(
                    dstp.at[pl.ds(pl.multiple_of(ebase + j * 128, 128), 128)],
                    dstvs[p][j], lsem)

        def drain_loads(ch, p):
            ebase = pl.multiple_of(s * ET + ch * CHUNK, CHUNK)
            sbase = pl.multiple_of(c * EPAD + s * ET + ch * CHUNK, CHUNK)
            pltpu.make_async_copy(
                srcp.at[pl.ds(sbase, CHUNK)], srcgs[p], lsem).wait()
            pltpu.make_async_copy(
                ew.at[pl.ds(ebase, CHUNK)], wvs[p], lsem).wait()
            for j in range(4):
                pltpu.make_async_copy(
                    dstp.at[pl.ds(pl.multiple_of(ebase + j * 128, 128), 128)],
                    dstvs[p][j], lsem).wait()

        def fire_gather(p, j):
            pltpu.async_copy(tbl_in.at[srcgs[p].at[pl.ds(j * 128, 128)]],
                             rows.at[pl.ds(j * 128, 128)], gsem)

        def drain_gather(p, j):
            pltpu.make_async_copy(
                tbl_in.at[srcgs[p].at[pl.ds(j * 128, 128)]],
                rows.at[pl.ds(j * 128, 128)], gsem).wait()

        def fire_scatter(p, j):
            pltpu.async_copy(rows.at[pl.ds(j * 128, 128)],
                             acc.at[dstvs[p][j]], ssem, add=True)

        def drain_scatter(p, j):
            pltpu.make_async_copy(rows.at[pl.ds(j * 128, 128)],
                                  acc.at[dstvs[p][j]], ssem).wait()

        def chunk_step(ch, p):
            nxt1 = jnp.minimum(ch + 1, NCH - 1)
            drain_loads(nxt1, 1 - p)
            for j in range(4):
                drain_gather(p, j)

                @plsc.parallel_loop(0, 8, unroll=2)
                def mul_body(g):
                    base = pl.multiple_of(j * 128 + g * 16, 16)
                    wvv = wvs[p][pl.ds(base, 16)]
                    for jj in range(16):
                        e = base + jj
                        wj = jnp.full((16,), wvv[jj], jnp.float32)
                        a = rows[e, pl.ds(0, 16)]
                        rows[e, pl.ds(0, 16)] = a * wj
                        b = rows[e, pl.ds(16, 16)]
                        rows[e, pl.ds(16, 16)] = b * wj

                fire_scatter(p, j)
                if j >= 1:
                    drain_scatter(p, j - 1)
                    fire_gather(1 - p, j - 1)
            drain_scatter(p, 3)
            fire_gather(1 - p, 3)
            fire_loads(jnp.minimum(ch + 2, NCH - 1), p)

        fire_loads(0, 0)
        drain_loads(0, 0)
        for j in range(4):
            fire_gather(0, j)
        fire_loads(1, 1)

        def pair_step(i, carry):
            chunk_step(2 * i, 0)
            chunk_step(2 * i + 1, 1)
            return carry

        lax.fori_loop(0, NCH // 2, pair_step, 0)
        for j in range(4):
            drain_gather(0, j)
        drain_loads(NCH - 1, 1)
        plsc.subcore_barrier()

    def writeback(dst_tbl, rezero):
        base = s * RPT
        coff = c * NPAD
        wdescs = [
            pltpu.async_copy(
                acc.at[pl.ds(pl.multiple_of(base + r * 128, 128), 128)],
                dst_tbl.at[pl.ds(pl.multiple_of(coff + base + r * 128, 128),
                                 128)],
                ssem)
            for r in range(RPT // 128)
        ]
        for dsc in wdescs:
            dsc.wait()
        if rezero:
            zero_zb()
            zdescs = [
                pltpu.async_copy(
                    zb,
                    acc.at[pl.ds(pl.multiple_of(base + r * 128, 128), 128)],
                    gsem)
                for r in range(RPT // 128)
            ]
            for dsc in zdescs:
                dsc.wait()
        plsc.subcore_barrier()

    layer(t0)
    writeback(t1, rezero=True)
    layer(t1)
    writeback(t2, rezero=False)

    pltpu.make_async_copy(uidx.at[pl.ds(gb0, PB)], ubuf, psem).wait()
    pltpu.make_async_copy(iidx.at[pl.ds(gb0, PB)], ibuf, psem).wait()

    def pair_body(q, cr):
        qs = pl.multiple_of(s * PB + q * PCH, PCH)
        qo = pl.multiple_of(q * PCH, PCH)
        ub = ubuf.at[pl.ds(qo, PCH)]
        ib = ibuf.at[pl.ds(qo, PCH)]

        descs = [
            pltpu.async_copy(t0.at[ub], rows.at[pl.ds(0, PCH)], gsem),
            pltpu.async_copy(t1.at[ub], rows.at[pl.ds(64, PCH)], gsem),
            pltpu.async_copy(t2.at[ub], rows.at[pl.ds(128, PCH)], gsem),
            pltpu.async_copy(t0.at[ib], rows.at[pl.ds(192, PCH)], gsem),
            pltpu.async_copy(t1.at[ib], rows.at[pl.ds(256, PCH)], gsem),
            pltpu.async_copy(t2.at[ib], rows.at[pl.ds(320, PCH)], gsem),
        ]
        for dsc in descs:
            dsc.wait()

        def sum_body(g, cr2):
            gb = g * 16
            for j in range(16):
                p = gb + j
                for h in (0, 16):
                    sl = pl.ds(h, 16)
                    rows[p, sl] = (rows[p, sl] + rows[64 + p, sl]
                                   + rows[128 + p, sl])
                    rows[192 + p, sl] = (rows[192 + p, sl]
                                         + rows[256 + p, sl]
                                         + rows[320 + p, sl])
            return cr2

        lax.fori_loop(0, PCH // 16, sum_body, 0)
        pltpu.sync_copy(rows.at[pl.ds(0, PCH)], usum.at[c, pl.ds(qs, PCH)])
        pltpu.sync_copy(rows.at[pl.ds(192, PCH)], isum.at[c, pl.ds(qs, PCH)])
        return cr

    lax.fori_loop(0, PB // PCH, pair_body, 0)


_sc_call = pl.kernel(
    _sc_body,
    out_type=[
        jax.ShapeDtypeStruct((2, B, HD), jnp.float32),
        jax.ShapeDtypeStruct((2, B, HD), jnp.float32),
        jax.ShapeDtypeStruct((2 * NPAD, HD), jnp.float32),
        jax.ShapeDtypeStruct((2 * NPAD, HD), jnp.float32),
    ],
    mesh=plsc.VectorSubcoreMesh(core_axis_name="c", subcore_axis_name="s"),
    compiler_params=pltpu.CompilerParams(use_tc_tiling_on_sc=False),
    scratch_types=[
        pltpu.VMEM_SHARED((NPAD, HD), jnp.float32),
        pltpu.VMEM((CHUNK, HD), jnp.float32),
        pltpu.VMEM((CHUNK,), jnp.int32),
        pltpu.VMEM((CHUNK,), jnp.int32),
        pltpu.VMEM((128,), jnp.int32),
        pltpu.VMEM((128,), jnp.int32),
        pltpu.VMEM((128,), jnp.int32),
        pltpu.VMEM((128,), jnp.int32),
        pltpu.VMEM((128,), jnp.int32),
        pltpu.VMEM((128,), jnp.int32),
        pltpu.VMEM((128,), jnp.int32),
        pltpu.VMEM((128,), jnp.int32),
        pltpu.VMEM((CHUNK,), jnp.float32),
        pltpu.VMEM((CHUNK,), jnp.float32),
        pltpu.VMEM((PB,), jnp.int32),
        pltpu.VMEM((PB,), jnp.int32),
        pltpu.SemaphoreType.DMA,
        pltpu.SemaphoreType.DMA,
        pltpu.SemaphoreType.DMA,
        pltpu.SemaphoreType.DMA,
    ],
)


def _loss_body(u_ref, i_ref, y_ref, out_ref):
    m = u_ref[0] * i_ref[0] + u_ref[1] * i_ref[1]
    l = jnp.sum(m, axis=-1) * (1.0 / 9.0)
    y = y_ref[...]
    sp = jnp.maximum(l, 0.0) + jnp.log1p(jnp.exp(-jnp.abs(l)))
    out_ref[0, 0] = jnp.sum(sp - l * y) * (1.0 / B)


_loss_call = pl.pallas_call(
    _loss_body,
    out_shape=jax.ShapeDtypeStruct((1, 1), jnp.float32),
    in_specs=[
        pl.BlockSpec(memory_space=pltpu.VMEM),
        pl.BlockSpec(memory_space=pltpu.VMEM),
        pl.BlockSpec(memory_space=pltpu.VMEM),
    ],
    out_specs=pl.BlockSpec(memory_space=pltpu.SMEM),
)


def kernel(user_emb, item_emb, edge_weight, users, items, labels, edge_index):
    all_emb = jnp.concatenate([user_emb, item_emb], axis=0)
    allp = jnp.concatenate(
        [all_emb, jnp.zeros((NPAD - NN, D), jnp.float32)], axis=0)
    t0 = jnp.concatenate([allp[:, :HD], allp[:, HD:]], axis=0)

    src_p0 = jnp.concatenate(
        [edge_index[0], jnp.zeros((EPAD - E,), jnp.int32)])
    src_p = jnp.concatenate([src_p0, src_p0 + NPAD])
    dst_p = jnp.concatenate(
        [edge_index[1], jnp.full((EPAD - E,), NN, jnp.int32)])
    ew_p = jnp.concatenate(
        [edge_weight, jnp.zeros((EPAD - E,), jnp.float32)])

    users_b = jnp.concatenate([users, users + NPAD])
    items_b = jnp.concatenate([items + NU, items + (NU + NPAD)])

    usum, isum, _, _ = _sc_call(t0, src_p, dst_p, ew_p, users_b, items_b)

    u4 = usum.reshape(2, 128, 128, HD)
    i4 = isum.reshape(2, 128, 128, HD)
    y = labels.astype(jnp.float32).reshape(128, 128)
    loss = _loss_call(u4, i4, y)
    return loss[0, 0]
